# baseline (speedup 1.0000x reference)
import functools

import jax
import jax.numpy as jnp
from jax import lax
from jax.experimental import pallas as pl
from jax.experimental.pallas import tpu as pltpu
from jax.experimental.pallas import tpu_sc as plsc

_B, _N, _C = 64, 576, 768
_K = 32
_NUM_CLASSES = 1000

_NC, _NS = 2, 16
_NW = _NC * _NS
_BH = _B // 2
_RH = _BH * _K
_NCH = _N // 16


@functools.cache
def _make_sc_topk_gather(sample_base):
    @functools.partial(
        pl.kernel,
        out_type=jax.ShapeDtypeStruct((_RH, _C), jnp.float32),
        mesh=plsc.VectorSubcoreMesh(
            core_axis_name="c", subcore_axis_name="s",
            num_cores=_NC, num_subcores=_NS,
        ),
        scratch_types=[
            pltpu.VMEM((_N,), jnp.float32),
            pltpu.VMEM((_K,), jnp.int32),
            pltpu.VMEM((_K, _C), jnp.float32),
            pltpu.SemaphoreType.DMA,
        ],
        compiler_params=pltpu.CompilerParams(needs_layout_passes=False),
    )
    def _sc_topk_gather(x_hbm, r_hbm, out_hbm, rv, idxv, rowsv, sem):
        wid = lax.axis_index("s") * _NC + lax.axis_index("c")
        iota = lax.iota(jnp.int32, 16)

        def _sort(v):
            return plsc.sort_key_val(v, v)[0]

        pltpu.sync_copy(r_hbm.at[sample_base + wid], rv)

        c0 = _sort(rv[pl.ds(0, 16)])
        c1 = _sort(rv[pl.ds(16, 16)])
        top = _sort(jnp.maximum(c0, jnp.flip(c1)))
        und = _sort(jnp.minimum(c0, jnp.flip(c1)))

        def merge(j, tu):
            top, und = tu
            c = _sort(rv[pl.ds(j * 16, 16)])
            hi = _sort(jnp.maximum(und, jnp.flip(c)))
            lo = jnp.minimum(und, jnp.flip(c))
            ntop = jnp.maximum(top, jnp.flip(hi))
            mid = _sort(jnp.minimum(top, jnp.flip(hi)))
            los = _sort(lo)
            nund = _sort(jnp.maximum(mid, jnp.flip(los)))
            return _sort(ntop), nund

        top, und = lax.fori_loop(2, _NCH, merge, (top, und))
        thr = jnp.full((16,), jnp.min(und), jnp.float32)

        gbase = (sample_base + wid) * _N
        start = jnp.zeros((16,), jnp.int32)
        end = jnp.full((16,), _K, jnp.int32)

        def strict(j, cnt):
            v = rv[pl.ds(j * 16, 16)]
            m = v > thr
            pos = plsc.cumsum(m.astype(jnp.int32)) - 1 + cnt
            plsc.store_scatter(idxv, [pos], iota + (j * 16 + gbase), mask=m)
            return cnt + plsc.all_reduce_population_count(m)

        cnt = lax.fori_loop(0, _NCH, strict, start)

        def ties(j, cnt):
            v = rv[pl.ds(j * 16, 16)]
            m = v == thr
            pos = plsc.cumsum(m.astype(jnp.int32)) - 1 + cnt
            sel = m & (pos < end)
            plsc.store_scatter(idxv, [pos], iota + (j * 16 + gbase), mask=sel)
            return cnt + plsc.all_reduce_population_count(sel)

        lax.fori_loop(0, _NCH, ties, cnt)

        pltpu.async_copy(x_hbm.at[idxv], rowsv, sem).wait()
        pltpu.sync_copy(rowsv, out_hbm.at[pl.ds(wid * _K, _K)])

    return _sc_topk_gather


def _dense_body(xt_ref, ww_ref, wb_ref, vw_ref, fcw_ref, fcb_ref, out_ref):
    xt = xt_ref[...]
    h = jnp.tanh(
        lax.dot_general(xt, ww_ref[...], (((1,), (1,)), ((), ())),
                        preferred_element_type=jnp.float32)
        + wb_ref[...]
    )
    s = lax.dot_general(h, vw_ref[...], (((1,), (1,)), ((), ())),
                        preferred_element_type=jnp.float32)
    e = jnp.exp(s - jnp.max(s))
    gcol = lax.broadcasted_iota(jnp.int32, (_BH, _RH), 1)
    grow = lax.broadcasted_iota(jnp.int32, (_BH, _RH), 0)
    g = (lax.div(gcol, jnp.int32(_K)) == grow).astype(jnp.float32)
    tcol = lax.broadcasted_iota(jnp.int32, (_RH, _BH), 1)
    trow = lax.broadcasted_iota(jnp.int32, (_RH, _BH), 0)
    gt = (lax.div(trow, jnp.int32(_K)) == tcol).astype(jnp.float32)
    gs = jnp.dot(g, e, preferred_element_type=jnp.float32)
    denom = jnp.dot(gt, gs, preferred_element_type=jnp.float32)
    w = xt * (e / denom)
    z = jnp.dot(g, w, preferred_element_type=jnp.float32)
    out_ref[...] = (
        lax.dot_general(z, fcw_ref[...], (((1,), (1,)), ((), ())),
                        preferred_element_type=jnp.float32)
        + fcb_ref[...]
    )


def _dense(xt, pool_W_w, pool_W_b2, pool_v_w, fc_w, fc_b2):
    return pl.pallas_call(
        _dense_body,
        out_shape=jax.ShapeDtypeStruct((_BH, _NUM_CLASSES), jnp.float32),
    )(xt, pool_W_w, pool_W_b2, pool_v_w, fc_w, fc_b2)


def kernel(x, r, pool_W_w, pool_W_b, pool_v_w, pool_v_b, fc_w, fc_b):
    x2d = x.reshape(_B * _N, _C)
    wb2 = pool_W_b.reshape(1, _C)
    fcb2 = fc_b.reshape(1, _NUM_CLASSES)
    xt0 = _make_sc_topk_gather(0)(x2d, r)
    xt1 = _make_sc_topk_gather(_BH)(x2d, r)
    l0 = _dense(xt0, pool_W_w, wb2, pool_v_w, fc_w, fcb2)
    l1 = _dense(xt1, pool_W_w, wb2, pool_v_w, fc_w, fcb2)
    return jnp.concatenate([l0, l1], axis=0)


# speedup vs baseline: 1.1005x; 1.1005x over previous
import functools

import jax
import jax.numpy as jnp
from jax import lax
from jax.experimental import pallas as pl
from jax.experimental.pallas import tpu as pltpu
from jax.experimental.pallas import tpu_sc as plsc

_B, _N, _C = 64, 576, 768
_K = 32
_NUM_CLASSES = 1000
_ROWS = _B * _K

_NC, _NS = 2, 16
_NW = _NC * _NS
_SPW = _B // _NW
_RPW = _ROWS // _NW
_NCH = _N // 16

_GB = 4
_RB = _ROWS // _GB
_BB = _RB // _K


@functools.cache
def _make_sc_topk_gather():
    @functools.partial(
        pl.kernel,
        out_type=jax.ShapeDtypeStruct((_ROWS, _C), jnp.float32),
        mesh=plsc.VectorSubcoreMesh(
            core_axis_name="c", subcore_axis_name="s",
            num_cores=_NC, num_subcores=_NS,
        ),
        scratch_types=[
            pltpu.VMEM((_N,), jnp.float32),
            pltpu.VMEM((_RPW,), jnp.int32),
            pltpu.VMEM((_RPW, _C), jnp.float32),
            pltpu.SemaphoreType.DMA,
        ],
        compiler_params=pltpu.CompilerParams(needs_layout_passes=False),
    )
    def _sc_topk_gather(x_hbm, r_hbm, out_hbm, rv, idxv, rowsv, sem):
        wid = lax.axis_index("s") * _NC + lax.axis_index("c")
        iota = lax.iota(jnp.int32, 16)

        def _sort(v):
            return plsc.sort_key_val(v, v)[0]

        for t in range(_SPW):
            b = wid * _SPW + t
            pltpu.sync_copy(r_hbm.at[b], rv)

            c0 = _sort(rv[pl.ds(0, 16)])
            c1 = _sort(rv[pl.ds(16, 16)])
            top = _sort(jnp.maximum(c0, jnp.flip(c1)))
            und = _sort(jnp.minimum(c0, jnp.flip(c1)))

            def merge(j, tu):
                top, und = tu
                c = _sort(rv[pl.ds(j * 16, 16)])
                hi = _sort(jnp.maximum(und, jnp.flip(c)))
                lo = jnp.minimum(und, jnp.flip(c))
                ntop = jnp.maximum(top, jnp.flip(hi))
                mid = _sort(jnp.minimum(top, jnp.flip(hi)))
                los = _sort(lo)
                nund = _sort(jnp.maximum(mid, jnp.flip(los)))
                return _sort(ntop), nund

            top, und = lax.fori_loop(2, _NCH, merge, (top, und))
            thr = jnp.full((16,), jnp.min(und), jnp.float32)

            gbase = b * _N
            start = jnp.full((16,), t * _K, jnp.int32)
            end = jnp.full((16,), t * _K + _K, jnp.int32)

            def strict(j, cnt):
                v = rv[pl.ds(j * 16, 16)]
                m = v > thr
                pos = plsc.cumsum(m.astype(jnp.int32)) - 1 + cnt
                plsc.store_scatter(idxv, [pos], iota + (j * 16 + gbase), mask=m)
                return cnt + plsc.all_reduce_population_count(m)

            cnt = lax.fori_loop(0, _NCH, strict, start)

            def ties(j, cnt):
                v = rv[pl.ds(j * 16, 16)]
                m = v == thr
                pos = plsc.cumsum(m.astype(jnp.int32)) - 1 + cnt
                sel = m & (pos < end)
                plsc.store_scatter(idxv, [pos], iota + (j * 16 + gbase),
                                   mask=sel)
                return cnt + plsc.all_reduce_population_count(sel)

            lax.fori_loop(0, _NCH, ties, cnt)

        pltpu.async_copy(x_hbm.at[idxv], rowsv, sem).wait()
        pltpu.sync_copy(rowsv, out_hbm.at[pl.ds(wid * _RPW, _RPW)])

    return _sc_topk_gather


def _dense_body(xt_ref, ww_ref, wb_ref, vw_ref, fcw_ref, fcb_ref, out_ref,
                u_ref, den_ref):
    i = pl.program_id(0)
    xt = xt_ref[...]
    h = jnp.tanh(
        lax.dot_general(xt, ww_ref[...], (((1,), (1,)), ((), ())),
                        preferred_element_type=jnp.float32)
        + wb_ref[...]
    )
    e = jnp.exp(
        lax.dot_general(h, vw_ref[...], (((1,), (1,)), ((), ())),
                        preferred_element_type=jnp.float32)
    )
    gcol = lax.broadcasted_iota(jnp.int32, (_BB, _RB), 1)
    grow = lax.broadcasted_iota(jnp.int32, (_BB, _RB), 0)
    g = (lax.div(gcol, jnp.int32(_K)) == grow).astype(jnp.float32)
    u_ref[pl.ds(i * _BB, _BB), :] = jnp.dot(g, xt * e,
                                            preferred_element_type=jnp.float32)
    den_ref[pl.ds(i * _BB, _BB), :] = jnp.dot(g, e,
                                              preferred_element_type=jnp.float32)

    @pl.when(i == _GB - 1)
    def _():
        z = u_ref[...] / den_ref[...]
        out_ref[...] = (
            lax.dot_general(z, fcw_ref[...], (((1,), (1,)), ((), ())),
                            preferred_element_type=jnp.float32)
            + fcb_ref[...]
        )


def _dense(xt, pool_W_w, pool_W_b, pool_v_w, fc_w, fc_b):
    return pl.pallas_call(
        _dense_body,
        grid=(_GB,),
        in_specs=[
            pl.BlockSpec((_RB, _C), lambda i: (i, 0)),
            pl.BlockSpec((_C, _C), lambda i: (0, 0)),
            pl.BlockSpec((1, _C), lambda i: (0, 0)),
            pl.BlockSpec((1, _C), lambda i: (0, 0)),
            pl.BlockSpec((_NUM_CLASSES, _C), lambda i: (0, 0)),
            pl.BlockSpec((1, _NUM_CLASSES), lambda i: (0, 0)),
        ],
        out_specs=pl.BlockSpec((_B, _NUM_CLASSES), lambda i: (0, 0)),
        out_shape=jax.ShapeDtypeStruct((_B, _NUM_CLASSES), jnp.float32),
        scratch_shapes=[
            pltpu.VMEM((_B, _C), jnp.float32),
            pltpu.VMEM((_B, 1), jnp.float32),
        ],
    )(
        xt,
        pool_W_w,
        pool_W_b.reshape(1, _C),
        pool_v_w,
        fc_w,
        fc_b.reshape(1, _NUM_CLASSES),
    )


def kernel(x, r, pool_W_w, pool_W_b, pool_v_w, pool_v_b, fc_w, fc_b):
    xt = _make_sc_topk_gather()(x.reshape(_B * _N, _C), r)
    return _dense(xt, pool_W_w, pool_W_b, pool_v_w, fc_w, fc_b)


# speedup vs baseline: 1.1319x; 1.0285x over previous
import functools

import jax
import jax.numpy as jnp
from jax import lax
from jax.experimental import pallas as pl
from jax.experimental.pallas import tpu as pltpu
from jax.experimental.pallas import tpu_sc as plsc

_B, _N, _C = 64, 576, 768
_K = 32
_NUM_CLASSES = 1000
_ROWS = _B * _K

_NC, _NS = 2, 16
_NW = _NC * _NS
_SPW = _B // _NW
_RPW = _ROWS // _NW
_NCH = _N // 16

_GB = 4
_RB = _ROWS // _GB
_BB = _RB // _K


@functools.cache
def _make_sc_topk_gather():
    @functools.partial(
        pl.kernel,
        out_type=jax.ShapeDtypeStruct((_ROWS, _C), jnp.float32),
        mesh=plsc.VectorSubcoreMesh(
            core_axis_name="c", subcore_axis_name="s",
            num_cores=_NC, num_subcores=_NS,
        ),
        scratch_types=[
            pltpu.VMEM((_N,), jnp.float32),
            pltpu.VMEM((_N,), jnp.float32),
            pltpu.VMEM((_RPW,), jnp.int32),
            pltpu.VMEM((_RPW, _C), jnp.float32),
            pltpu.SemaphoreType.DMA,
        ],
        compiler_params=pltpu.CompilerParams(needs_layout_passes=False),
    )
    def _sc_topk_gather(x_hbm, r_hbm, out_hbm, rv0, rv1, idxv, rowsv, sem):
        wid = lax.axis_index("s") * _NC + lax.axis_index("c")
        iota = lax.iota(jnp.int32, 16)

        def _sort(v):
            return plsc.sort_key_val(v, v)[0]

        b0 = wid * _SPW
        b1 = b0 + 1
        pltpu.sync_copy(r_hbm.at[b0], rv0)
        pltpu.sync_copy(r_hbm.at[b1], rv1)
        rvs = (rv0, rv1)

        def _init(rv):
            c0 = _sort(rv[pl.ds(0, 16)])
            c1 = _sort(rv[pl.ds(16, 16)])
            return (_sort(jnp.maximum(c0, jnp.flip(c1))),
                    _sort(jnp.minimum(c0, jnp.flip(c1))))

        def _merge1(rv, j, top, und):
            c = _sort(rv[pl.ds(j * 16, 16)])
            hi = _sort(jnp.maximum(und, jnp.flip(c)))
            lo = jnp.minimum(und, jnp.flip(c))
            ntop = jnp.maximum(top, jnp.flip(hi))
            mid = _sort(jnp.minimum(top, jnp.flip(hi)))
            los = _sort(lo)
            nund = _sort(jnp.maximum(mid, jnp.flip(los)))
            return _sort(ntop), nund

        t0, u0 = _init(rv0)
        t1, u1 = _init(rv1)

        def merge(j, st):
            t0, u0, t1, u1 = st
            t0, u0 = _merge1(rv0, j, t0, u0)
            t1, u1 = _merge1(rv1, j, t1, u1)
            return t0, u0, t1, u1

        _, u0, _, u1 = lax.fori_loop(2, _NCH, merge, (t0, u0, t1, u1))
        thrs = (jnp.full((16,), jnp.min(u0), jnp.float32),
                jnp.full((16,), jnp.min(u1), jnp.float32))

        gbases = (b0 * _N, b1 * _N)
        start = (jnp.zeros((16,), jnp.int32),
                 jnp.full((16,), _K, jnp.int32))
        end = (jnp.full((16,), _K, jnp.int32),
               jnp.full((16,), 2 * _K, jnp.int32))

        def strict(j, cnts):
            out = []
            for s in range(_SPW):
                v = rvs[s][pl.ds(j * 16, 16)]
                m = v > thrs[s]
                pos = plsc.cumsum(m.astype(jnp.int32)) - 1 + cnts[s]
                plsc.store_scatter(idxv, [pos], iota + (j * 16 + gbases[s]),
                                   mask=m)
                out.append(cnts[s] + plsc.all_reduce_population_count(m))
            return tuple(out)

        cnts = lax.fori_loop(0, _NCH, strict, start)

        def ties(j, cnts):
            out = []
            for s in range(_SPW):
                v = rvs[s][pl.ds(j * 16, 16)]
                m = v == thrs[s]
                pos = plsc.cumsum(m.astype(jnp.int32)) - 1 + cnts[s]
                sel = m & (pos < end[s])
                plsc.store_scatter(idxv, [pos], iota + (j * 16 + gbases[s]),
                                   mask=sel)
                out.append(cnts[s] + plsc.all_reduce_population_count(sel))
            return tuple(out)

        lax.fori_loop(0, _NCH, ties, cnts)

        pltpu.async_copy(x_hbm.at[idxv], rowsv, sem).wait()
        pltpu.sync_copy(rowsv, out_hbm.at[pl.ds(wid * _RPW, _RPW)])

    return _sc_topk_gather


def _dense_body(xt_ref, ww_ref, wb_ref, vw_ref, fcw_ref, fcb_ref, out_ref,
                u_ref, den_ref):
    i = pl.program_id(0)
    xt = xt_ref[...]
    h = jnp.tanh(
        lax.dot_general(xt.astype(jnp.bfloat16),
                        ww_ref[...].astype(jnp.bfloat16),
                        (((1,), (1,)), ((), ())),
                        preferred_element_type=jnp.float32)
        + wb_ref[...]
    )
    e = jnp.exp(
        lax.dot_general(h, vw_ref[...], (((1,), (1,)), ((), ())),
                        preferred_element_type=jnp.float32)
    )
    gcol = lax.broadcasted_iota(jnp.int32, (_BB, _RB), 1)
    grow = lax.broadcasted_iota(jnp.int32, (_BB, _RB), 0)
    g = (lax.div(gcol, jnp.int32(_K)) == grow).astype(jnp.float32)
    u_ref[pl.ds(i * _BB, _BB), :] = jnp.dot(g, xt * e,
                                            preferred_element_type=jnp.float32)
    den_ref[pl.ds(i * _BB, _BB), :] = jnp.dot(g, e,
                                              preferred_element_type=jnp.float32)

    @pl.when(i == _GB - 1)
    def _():
        z = u_ref[...] / den_ref[...]
        out_ref[...] = (
            lax.dot_general(z, fcw_ref[...], (((1,), (1,)), ((), ())),
                            preferred_element_type=jnp.float32)
            + fcb_ref[...]
        )


def _dense(xt, pool_W_w, pool_W_b, pool_v_w, fc_w, fc_b):
    return pl.pallas_call(
        _dense_body,
        grid=(_GB,),
        in_specs=[
            pl.BlockSpec((_RB, _C), lambda i: (i, 0)),
            pl.BlockSpec((_C, _C), lambda i: (0, 0)),
            pl.BlockSpec((1, _C), lambda i: (0, 0)),
            pl.BlockSpec((1, _C), lambda i: (0, 0)),
            pl.BlockSpec((_NUM_CLASSES, _C), lambda i: (0, 0)),
            pl.BlockSpec((1, _NUM_CLASSES), lambda i: (0, 0)),
        ],
        out_specs=pl.BlockSpec((_B, _NUM_CLASSES), lambda i: (0, 0)),
        out_shape=jax.ShapeDtypeStruct((_B, _NUM_CLASSES), jnp.float32),
        scratch_shapes=[
            pltpu.VMEM((_B, _C), jnp.float32),
            pltpu.VMEM((_B, 1), jnp.float32),
        ],
    )(
        xt,
        pool_W_w,
        pool_W_b.reshape(1, _C),
        pool_v_w,
        fc_w,
        fc_b.reshape(1, _NUM_CLASSES),
    )


def kernel(x, r, pool_W_w, pool_W_b, pool_v_w, pool_v_b, fc_w, fc_b):
    xt = _make_sc_topk_gather()(x.reshape(_B * _N, _C), r)
    return _dense(xt, pool_W_w, pool_W_b, pool_v_w, fc_w, fc_b)
